# baseline (speedup 1.0000x reference)
import functools

import jax
import jax.numpy as jnp
from jax import lax
from jax.experimental import pallas as pl
from jax.experimental.pallas import tpu as pltpu
from jax.experimental.pallas import tpu_sc as plsc

B = 4096
H = 50
C = 64
L = 16
NC, NS = 2, 16
NW = NC * NS
RPW = B // NW
CB = 2
K = CB * H
NCH = RPW // CB
G = C // L


def _body(x_hbm, w_hbm, table_hbm, out_hbm, idx_v, w_v, rows0, rows1, out_v,
          sem0, sem1):
    wid = lax.axis_index("s") * NC + lax.axis_index("c")

    pltpu.sync_copy(x_hbm.at[wid], idx_v)
    pltpu.sync_copy(w_hbm.at[wid], w_v)

    sems = (sem0, sem1)
    rows = (rows0, rows1)

    def start(j, slot):
        pltpu.async_copy(table_hbm.at[idx_v.at[j]], rows[slot], sems[slot])

    def wait(j, slot):
        pltpu.make_async_copy(table_hbm.at[idx_v.at[j]], rows[slot],
                              sems[slot]).wait()

    def compute(j, slot):
        rv = rows[slot]
        fj = jnp.full((L,), j * K, jnp.int32)
        for r in range(CB):
            acc = [jnp.zeros((L,), jnp.float32) for _ in range(G)]
            for l in range(H):
                p = r * H + l
                ws = plsc.load_gather(w_v, [fj + p])
                for g in range(G):
                    acc[g] = acc[g] + ws * rv[p, pl.ds(g * L, L)]
            row = CB * j + r
            for g in range(G):
                out_v[row, pl.ds(g * L, L)] = acc[g]

    @pl.loop(0, NCH)
    def _(j):
        start(j, 0)
        wait(j, 0)
        compute(j, 0)

    pltpu.sync_copy(out_v, out_hbm.at[pl.ds(wid * RPW, RPW)])


@jax.jit
def kernel(x, w, table):
    xr = x.reshape(NW, NCH, K).astype(jnp.int32)
    wr = w.reshape(NW, NCH * K)
    mesh = plsc.VectorSubcoreMesh(core_axis_name="c", subcore_axis_name="s")
    f = pl.kernel(
        functools.partial(_body),
        out_type=jax.ShapeDtypeStruct((B, C), jnp.float32),
        mesh=mesh,
        compiler_params=pltpu.CompilerParams(
            needs_layout_passes=False, use_tc_tiling_on_sc=False),
        scratch_types=[
            pltpu.VMEM((NCH, K), jnp.int32),
            pltpu.VMEM((NCH * K,), jnp.float32),
            pltpu.VMEM((K, C), jnp.float32),
            pltpu.VMEM((K, C), jnp.float32),
            pltpu.VMEM((RPW, C), jnp.float32),
            pltpu.SemaphoreType.DMA,
            pltpu.SemaphoreType.DMA,
        ],
    )
    return f(xr, wr, table)


# speedup vs baseline: 1.0535x; 1.0535x over previous
import functools

import jax
import jax.numpy as jnp
from jax import lax
from jax.experimental import pallas as pl
from jax.experimental.pallas import tpu as pltpu
from jax.experimental.pallas import tpu_sc as plsc

B = 4096
H = 50
C = 64
L = 16
NC, NS = 2, 16
NW = NC * NS
RPW = B // NW
CB = 2
K = CB * H
NCH = RPW // CB
G = C // L


def _body(x_hbm, w_hbm, table_hbm, out_hbm, idx_v, w_v, rows0, rows1, out_v,
          sem0, sem1):
    wid = lax.axis_index("s") * NC + lax.axis_index("c")

    pltpu.sync_copy(x_hbm.at[wid], idx_v)
    pltpu.sync_copy(w_hbm.at[wid], w_v)

    sems = (sem0, sem1)
    rows = (rows0, rows1)

    def start(j, slot):
        pltpu.async_copy(table_hbm.at[idx_v.at[j]], rows[slot], sems[slot])

    def wait(j, slot):
        pltpu.make_async_copy(table_hbm.at[idx_v.at[j]], rows[slot],
                              sems[slot]).wait()

    def compute(j, slot):
        rv = rows[slot]
        fj = jnp.full((L,), j * K, jnp.int32)
        for r in range(CB):
            acc = [jnp.zeros((L,), jnp.float32) for _ in range(G)]
            for l in range(H):
                p = r * H + l
                ws = plsc.load_gather(w_v, [fj + p])
                for g in range(G):
                    acc[g] = acc[g] + ws * rv[p, pl.ds(g * L, L)]
            row = CB * j + r
            for g in range(G):
                out_v[row, pl.ds(g * L, L)] = acc[g]

    start(0, 0)

    @pl.loop(0, NCH // 2)
    def _(t):
        j0 = 2 * t
        j1 = j0 + 1
        start(j1, 1)
        wait(j0, 0)
        compute(j0, 0)
        start(lax.rem(j0 + 2, NCH), 0)
        wait(j1, 1)
        compute(j1, 1)

    wait(0, 0)

    pltpu.sync_copy(out_v, out_hbm.at[pl.ds(wid * RPW, RPW)])


@jax.jit
def kernel(x, w, table):
    xr = x.reshape(NW, NCH, K).astype(jnp.int32)
    wr = w.reshape(NW, NCH * K)
    mesh = plsc.VectorSubcoreMesh(core_axis_name="c", subcore_axis_name="s")
    f = pl.kernel(
        functools.partial(_body),
        out_type=jax.ShapeDtypeStruct((B, C), jnp.float32),
        mesh=mesh,
        compiler_params=pltpu.CompilerParams(
            needs_layout_passes=False, use_tc_tiling_on_sc=False),
        scratch_types=[
            pltpu.VMEM((NCH, K), jnp.int32),
            pltpu.VMEM((NCH * K,), jnp.float32),
            pltpu.VMEM((K, C), jnp.float32),
            pltpu.VMEM((K, C), jnp.float32),
            pltpu.VMEM((RPW, C), jnp.float32),
            pltpu.SemaphoreType.DMA,
            pltpu.SemaphoreType.DMA,
        ],
    )
    return f(xr, wr, table)
